# baseline (speedup 1.0000x reference)
import functools

import jax
import jax.numpy as jnp
from jax import lax
from jax.experimental import pallas as pl
from jax.experimental.pallas import tpu as pltpu
from jax.experimental.pallas import tpu_sc as plsc

_INTERPRET = False

N_NODES = 10000
N_EDGES = 160000
D = 128


def _mm_bias_body(x_ref, w_ref, b_ref, o_ref):
    o_ref[...] = (
        jnp.dot(x_ref[...], w_ref[...], preferred_element_type=jnp.float32)
        + b_ref[...]
    )


def _mm_bias(x, w, b, blk):
    n, k = x.shape
    m = w.shape[1]
    grid = n // blk
    return pl.pallas_call(
        _mm_bias_body,
        grid=(grid,),
        in_specs=[
            pl.BlockSpec((blk, k), lambda i: (i, 0)),
            pl.BlockSpec((k, m), lambda i: (0, 0)),
            pl.BlockSpec((1, m), lambda i: (0, 0)),
        ],
        out_specs=pl.BlockSpec((blk, m), lambda i: (i, 0)),
        out_shape=jax.ShapeDtypeStruct((n, m), jnp.float32),
        interpret=_INTERPRET,
    )(x, w, b.reshape(1, m))


def _proj_body(n_ref, ws_ref, wr_ref, g_ref, wg_ref, b1_ref, ev_ref, we_ref,
               ps_ref, pr_ref):
    c = (
        jnp.dot(g_ref[...], wg_ref[...], preferred_element_type=jnp.float32)
        + b1_ref[...]
        + jnp.dot(ev_ref[...], we_ref[...], preferred_element_type=jnp.float32)
    )
    nb = n_ref[...]
    ps_ref[...] = (
        jnp.dot(nb, ws_ref[...], preferred_element_type=jnp.float32) + c
    )
    pr_ref[...] = jnp.dot(nb, wr_ref[...], preferred_element_type=jnp.float32)


def _proj(n, w1s, w1r, g, w1g, b1, extra_vec, extra_mat, blk=2000):
    nn = n.shape[0]
    grid = nn // blk
    return pl.pallas_call(
        _proj_body,
        grid=(grid,),
        in_specs=[
            pl.BlockSpec((blk, D), lambda i: (i, 0)),
            pl.BlockSpec((D, D), lambda i: (0, 0)),
            pl.BlockSpec((D, D), lambda i: (0, 0)),
            pl.BlockSpec((1, D), lambda i: (0, 0)),
            pl.BlockSpec((D, D), lambda i: (0, 0)),
            pl.BlockSpec((1, D), lambda i: (0, 0)),
            pl.BlockSpec((1, extra_mat.shape[0]), lambda i: (0, 0)),
            pl.BlockSpec((extra_mat.shape[0], D), lambda i: (0, 0)),
        ],
        out_specs=[
            pl.BlockSpec((blk, D), lambda i: (i, 0)),
            pl.BlockSpec((blk, D), lambda i: (i, 0)),
        ],
        out_shape=[
            jax.ShapeDtypeStruct((nn, D), jnp.float32),
            jax.ShapeDtypeStruct((nn, D), jnp.float32),
        ],
        interpret=_INTERPRET,
    )(n, w1s, w1r, g, w1g, b1.reshape(1, D), extra_vec, extra_mat)


def _edge_body(x_ref, g_ref, wa_ref, wb_ref, w2_ref, b2_ref,
               out_ref, agg_ref, acc_ref):
    a = jnp.dot(wa_ref[...], wb_ref[...], preferred_element_type=jnp.float32)
    h = (
        jnp.dot(x_ref[...], a, preferred_element_type=jnp.float32)
        + g_ref[...]
    )
    y = (
        jnp.dot(jnp.maximum(h, 0.0), w2_ref[...],
                preferred_element_type=jnp.float32)
        + b2_ref[...]
    )
    out_ref[...] = y
    i = pl.program_id(0)

    @pl.when(i == 0)
    def _():
        acc_ref[...] = jnp.zeros_like(acc_ref)

    acc_ref[...] += jnp.sum(y, axis=0, keepdims=True)

    @pl.when(i == pl.num_programs(0) - 1)
    def _():
        agg_ref[...] = acc_ref[...]


def _edge_mlp(x, g_rows, wa, wb, w2, b2, blk=2000):
    ne, k = x.shape
    grid = ne // blk
    return pl.pallas_call(
        _edge_body,
        grid=(grid,),
        in_specs=[
            pl.BlockSpec((blk, k), lambda i: (i, 0)),
            pl.BlockSpec((blk, D), lambda i: (i, 0)),
            pl.BlockSpec((k, D), lambda i: (0, 0)),
            pl.BlockSpec((D, D), lambda i: (0, 0)),
            pl.BlockSpec((D, D), lambda i: (0, 0)),
            pl.BlockSpec((1, D), lambda i: (0, 0)),
        ],
        out_specs=[
            pl.BlockSpec((blk, D), lambda i: (i, 0)),
            pl.BlockSpec((1, D), lambda i: (0, 0)),
        ],
        out_shape=[
            jax.ShapeDtypeStruct((ne, D), jnp.float32),
            jax.ShapeDtypeStruct((1, D), jnp.float32),
        ],
        scratch_shapes=[pltpu.VMEM((1, D), jnp.float32)],
        interpret=_INTERPRET,
    )(x, g_rows, wa, wb, w2, b2.reshape(1, D))


def _node_body(n_ref, s_ref, r_ref, vn_ref, vs_ref, vr_ref, g_ref, vg_ref,
               b1_ref, v2_ref, b2_ref, out_ref, agg_ref, acc_ref):
    c = (
        jnp.dot(g_ref[...], vg_ref[...], preferred_element_type=jnp.float32)
        + b1_ref[...]
    )
    h = (
        jnp.dot(n_ref[...], vn_ref[...], preferred_element_type=jnp.float32)
        + jnp.dot(s_ref[...], vs_ref[...], preferred_element_type=jnp.float32)
        + jnp.dot(r_ref[...], vr_ref[...], preferred_element_type=jnp.float32)
        + c
    )
    y = (
        jnp.dot(jnp.maximum(h, 0.0), v2_ref[...],
                preferred_element_type=jnp.float32)
        + b2_ref[...]
    )
    out_ref[...] = y
    i = pl.program_id(0)

    @pl.when(i == 0)
    def _():
        acc_ref[...] = jnp.zeros_like(acc_ref)

    acc_ref[...] += jnp.sum(y, axis=0, keepdims=True)

    @pl.when(i == pl.num_programs(0) - 1)
    def _():
        agg_ref[...] = acc_ref[...]


def _node_mlp(n, sent, recv, vn, vs, vr, g, vg, b1, v2, b2, blk=2000):
    nn = n.shape[0]
    grid = nn // blk
    full = lambda i: (0, 0)
    rows = lambda i: (i, 0)
    return pl.pallas_call(
        _node_body,
        grid=(grid,),
        in_specs=[
            pl.BlockSpec((blk, D), rows),
            pl.BlockSpec((blk, D), rows),
            pl.BlockSpec((blk, D), rows),
            pl.BlockSpec((D, D), full),
            pl.BlockSpec((D, D), full),
            pl.BlockSpec((D, D), full),
            pl.BlockSpec((1, D), full),
            pl.BlockSpec((D, D), full),
            pl.BlockSpec((1, D), full),
            pl.BlockSpec((D, D), full),
            pl.BlockSpec((1, D), full),
        ],
        out_specs=[
            pl.BlockSpec((blk, D), rows),
            pl.BlockSpec((1, D), full),
        ],
        out_shape=[
            jax.ShapeDtypeStruct((nn, D), jnp.float32),
            jax.ShapeDtypeStruct((1, D), jnp.float32),
        ],
        scratch_shapes=[pltpu.VMEM((1, D), jnp.float32)],
        interpret=_INTERPRET,
    )(n, sent, recv, vn, vs, vr, g, vg, b1.reshape(1, D), v2, b2.reshape(1, D))


def _glob_body(na_ref, ea_ref, g_ref, un_ref, ue_ref, ug_ref, b1_ref,
               u2_ref, b2_ref, o_ref):
    h = (
        jnp.dot(na_ref[...], un_ref[...], preferred_element_type=jnp.float32)
        + jnp.dot(ea_ref[...], ue_ref[...], preferred_element_type=jnp.float32)
        + jnp.dot(g_ref[...], ug_ref[...], preferred_element_type=jnp.float32)
        + b1_ref[...]
    )
    o_ref[...] = (
        jnp.dot(jnp.maximum(h, 0.0), u2_ref[...],
                preferred_element_type=jnp.float32)
        + b2_ref[...]
    )


def _glob_mlp(na, ea, g, un, ue, ug, b1, u2, b2):
    full = lambda: (0, 0)
    return pl.pallas_call(
        _glob_body,
        in_specs=[pl.BlockSpec(s, None) for s in
                  [(1, D), (1, D), (1, D), (D, D), (D, D), (D, D), (1, D),
                   (D, D), (1, D)]],
        out_specs=pl.BlockSpec((1, D), None),
        out_shape=jax.ShapeDtypeStruct((1, D), jnp.float32),
        interpret=_INTERPRET,
    )(na, ea, g, un, ue, ug, b1.reshape(1, D), u2, b2.reshape(1, D))


def _readout_body(g_ref, w1_ref, b1_ref, w2t_ref, b2_ref, o_ref):
    h = (
        jnp.dot(g_ref[...], w1_ref[...], preferred_element_type=jnp.float32)
        + b1_ref[...]
    )
    h = jnp.maximum(h, 0.0)
    o_ref[...] = (
        jnp.sum(h * w2t_ref[...], axis=1, keepdims=True) + b2_ref[...]
    )


def _readout(g, w1, b1, w2, b2):
    return pl.pallas_call(
        _readout_body,
        in_specs=[pl.BlockSpec(s, None) for s in
                  [(1, D), (D, D), (1, D), (1, D), (1, 1)]],
        out_specs=pl.BlockSpec((1, 1), None),
        out_shape=jax.ShapeDtypeStruct((1, 1), jnp.float32),
        interpret=_INTERPRET,
    )(g, w1, b1.reshape(1, D), w2.reshape(1, D), b2.reshape(1, 1))


def _sc_gather_sum(ps, pr, senders, receivers):
    return jnp.take(ps, senders, axis=0) + jnp.take(pr, receivers, axis=0)


def _sc_segment_sums(e, senders, receivers):
    sent = jax.ops.segment_sum(e, senders, num_segments=N_NODES)
    recv = jax.ops.segment_sum(e, receivers, num_segments=N_NODES)
    return sent, recv


def _softplus(x):
    return jnp.log(1.0 + jnp.exp(x))


def _sample_mlp(layers, key):
    ks = jax.random.split(key, len(layers))
    out = []
    for p, k in zip(layers, ks):
        w = p['w_mu'] + jax.random.normal(k, p['w_mu'].shape,
                                          dtype=jnp.float32) * _softplus(p['w_rho'])
        b = p['b_mu'] + jax.random.normal(k, p['b_mu'].shape,
                                          dtype=jnp.float32) * _softplus(p['b_rho'])
        out.append((w, b))
    return out


def kernel(nodes, edges, senders, receivers, globals_, positions, box, params):
    keys = jax.random.split(jax.random.key(42), 4)
    emb = params['embed']

    n = _mm_bias(nodes, emb['node_w'], emb['node_b'], blk=2000)
    g = _mm_bias(globals_, emb['glob_w'], emb['glob_b'], blk=1)

    e = None
    zero_vec = jnp.zeros((1, D), jnp.float32)
    eye = jnp.eye(D, dtype=jnp.float32)

    for s in range(2):
        sp = params['steps'][s]
        k_e, k_n, k_g = jax.random.split(keys[s], 3)
        (ew1, eb1), (ew2, eb2) = _sample_mlp(sp['edge'], k_e)
        (nw1, nb1), (nw2, nb2) = _sample_mlp(sp['node'], k_n)
        (gw1, gb1), (gw2, gb2) = _sample_mlp(sp['glob'], k_g)

        w1e = ew1[0:D]
        w1s = ew1[D:2 * D]
        w1r = ew1[2 * D:3 * D]
        w1g = ew1[3 * D:4 * D]

        if s == 0:
            ps, pr = _proj(n, w1s, w1r, g, w1g, eb1,
                           emb['edge_b'].reshape(1, D), w1e)
            x, wa, wb = edges, emb['edge_w'], w1e
        else:
            ps, pr = _proj(n, w1s, w1r, g, w1g, eb1, zero_vec, eye)
            x, wa, wb = e, w1e, eye

        grows = _sc_gather_sum(ps, pr, senders, receivers)
        e, e_agg = _edge_mlp(x, grows, wa, wb, ew2, eb2)
        sent, recv = _sc_segment_sums(e, senders, receivers)

        n, n_agg = _node_mlp(n, sent, recv,
                             nw1[0:D], nw1[D:2 * D], nw1[2 * D:3 * D],
                             g, nw1[3 * D:4 * D], nb1, nw2, nb2)
        g = _glob_mlp(n_agg, e_agg, g,
                      gw1[0:D], gw1[D:2 * D], gw1[2 * D:3 * D], gb1,
                      gw2, gb2)

    (rw1, rb1), (rw2, rb2) = _sample_mlp(params['readout'], keys[-1])
    return _readout(g, rw1, rb1, rw2, rb2)


# speedup vs baseline: 2.8680x; 2.8680x over previous
import functools

import jax
import jax.numpy as jnp
from jax import lax
from jax.experimental import pallas as pl
from jax.experimental.pallas import tpu as pltpu
from jax.experimental.pallas import tpu_sc as plsc

_INTERPRET = False

N_NODES = 10000
N_EDGES = 160000
D = 128


def _mm_bias_body(x_ref, w_ref, b_ref, o_ref):
    o_ref[...] = (
        jnp.dot(x_ref[...], w_ref[...], preferred_element_type=jnp.float32)
        + b_ref[...]
    )


def _mm_bias(x, w, b, blk):
    n, k = x.shape
    m = w.shape[1]
    grid = n // blk
    return pl.pallas_call(
        _mm_bias_body,
        grid=(grid,),
        in_specs=[
            pl.BlockSpec((blk, k), lambda i: (i, 0)),
            pl.BlockSpec((k, m), lambda i: (0, 0)),
            pl.BlockSpec((1, m), lambda i: (0, 0)),
        ],
        out_specs=pl.BlockSpec((blk, m), lambda i: (i, 0)),
        out_shape=jax.ShapeDtypeStruct((n, m), jnp.float32),
        interpret=_INTERPRET,
    )(x, w, b.reshape(1, m))


def _proj_body(n_ref, ws_ref, wr_ref, g_ref, wg_ref, b1_ref, ev_ref, we_ref,
               ps_ref, pr_ref):
    c = (
        jnp.dot(g_ref[...], wg_ref[...], preferred_element_type=jnp.float32)
        + b1_ref[...]
        + jnp.dot(ev_ref[...], we_ref[...], preferred_element_type=jnp.float32)
    )
    nb = n_ref[...]
    ps_ref[...] = (
        jnp.dot(nb, ws_ref[...], preferred_element_type=jnp.float32) + c
    )
    pr_ref[...] = jnp.dot(nb, wr_ref[...], preferred_element_type=jnp.float32)


def _proj(n, w1s, w1r, g, w1g, b1, extra_vec, extra_mat, blk=2000):
    nn = n.shape[0]
    grid = nn // blk
    return pl.pallas_call(
        _proj_body,
        grid=(grid,),
        in_specs=[
            pl.BlockSpec((blk, D), lambda i: (i, 0)),
            pl.BlockSpec((D, D), lambda i: (0, 0)),
            pl.BlockSpec((D, D), lambda i: (0, 0)),
            pl.BlockSpec((1, D), lambda i: (0, 0)),
            pl.BlockSpec((D, D), lambda i: (0, 0)),
            pl.BlockSpec((1, D), lambda i: (0, 0)),
            pl.BlockSpec((1, extra_mat.shape[0]), lambda i: (0, 0)),
            pl.BlockSpec((extra_mat.shape[0], D), lambda i: (0, 0)),
        ],
        out_specs=[
            pl.BlockSpec((blk, D), lambda i: (i, 0)),
            pl.BlockSpec((blk, D), lambda i: (i, 0)),
        ],
        out_shape=[
            jax.ShapeDtypeStruct((nn, D), jnp.float32),
            jax.ShapeDtypeStruct((nn, D), jnp.float32),
        ],
        interpret=_INTERPRET,
    )(n, w1s, w1r, g, w1g, b1.reshape(1, D), extra_vec, extra_mat)


def _edge_body(x_ref, gs_ref, gr_ref, wa_ref, wb_ref, w2_ref, b2_ref,
               out_ref, agg_ref, acc_ref):
    a = jnp.dot(wa_ref[...], wb_ref[...], preferred_element_type=jnp.float32)
    h = (
        jnp.dot(x_ref[...], a, preferred_element_type=jnp.float32)
        + gs_ref[...]
        + gr_ref[...]
    )
    y = (
        jnp.dot(jnp.maximum(h, 0.0), w2_ref[...],
                preferred_element_type=jnp.float32)
        + b2_ref[...]
    )
    out_ref[...] = y
    i = pl.program_id(0)

    @pl.when(i == 0)
    def _():
        acc_ref[...] = jnp.zeros_like(acc_ref)

    acc_ref[...] += jnp.sum(y, axis=0, keepdims=True)

    @pl.when(i == pl.num_programs(0) - 1)
    def _():
        agg_ref[...] = acc_ref[...]


def _edge_mlp(x, gs_rows, gr_rows, wa, wb, w2, b2, blk=2000):
    ne, k = x.shape
    grid = ne // blk
    return pl.pallas_call(
        _edge_body,
        grid=(grid,),
        in_specs=[
            pl.BlockSpec((blk, k), lambda i: (i, 0)),
            pl.BlockSpec((blk, D), lambda i: (i, 0)),
            pl.BlockSpec((blk, D), lambda i: (i, 0)),
            pl.BlockSpec((k, D), lambda i: (0, 0)),
            pl.BlockSpec((D, D), lambda i: (0, 0)),
            pl.BlockSpec((D, D), lambda i: (0, 0)),
            pl.BlockSpec((1, D), lambda i: (0, 0)),
        ],
        out_specs=[
            pl.BlockSpec((blk, D), lambda i: (i, 0)),
            pl.BlockSpec((1, D), lambda i: (0, 0)),
        ],
        out_shape=[
            jax.ShapeDtypeStruct((ne, D), jnp.float32),
            jax.ShapeDtypeStruct((1, D), jnp.float32),
        ],
        scratch_shapes=[pltpu.VMEM((1, D), jnp.float32)],
        interpret=_INTERPRET,
    )(x, gs_rows, gr_rows, wa, wb, w2, b2.reshape(1, D))


def _node_body(n_ref, s_ref, r_ref, vn_ref, vs_ref, vr_ref, g_ref, vg_ref,
               b1_ref, v2_ref, b2_ref, out_ref, agg_ref, acc_ref):
    c = (
        jnp.dot(g_ref[...], vg_ref[...], preferred_element_type=jnp.float32)
        + b1_ref[...]
    )
    h = (
        jnp.dot(n_ref[...], vn_ref[...], preferred_element_type=jnp.float32)
        + jnp.dot(s_ref[...], vs_ref[...], preferred_element_type=jnp.float32)
        + jnp.dot(r_ref[...], vr_ref[...], preferred_element_type=jnp.float32)
        + c
    )
    y = (
        jnp.dot(jnp.maximum(h, 0.0), v2_ref[...],
                preferred_element_type=jnp.float32)
        + b2_ref[...]
    )
    out_ref[...] = y
    i = pl.program_id(0)

    @pl.when(i == 0)
    def _():
        acc_ref[...] = jnp.zeros_like(acc_ref)

    acc_ref[...] += jnp.sum(y, axis=0, keepdims=True)

    @pl.when(i == pl.num_programs(0) - 1)
    def _():
        agg_ref[...] = acc_ref[...]


def _node_mlp(n, sent, recv, vn, vs, vr, g, vg, b1, v2, b2, blk=2000):
    nn = n.shape[0]
    grid = nn // blk
    full = lambda i: (0, 0)
    rows = lambda i: (i, 0)
    return pl.pallas_call(
        _node_body,
        grid=(grid,),
        in_specs=[
            pl.BlockSpec((blk, D), rows),
            pl.BlockSpec((blk, D), rows),
            pl.BlockSpec((blk, D), rows),
            pl.BlockSpec((D, D), full),
            pl.BlockSpec((D, D), full),
            pl.BlockSpec((D, D), full),
            pl.BlockSpec((1, D), full),
            pl.BlockSpec((D, D), full),
            pl.BlockSpec((1, D), full),
            pl.BlockSpec((D, D), full),
            pl.BlockSpec((1, D), full),
        ],
        out_specs=[
            pl.BlockSpec((blk, D), rows),
            pl.BlockSpec((1, D), full),
        ],
        out_shape=[
            jax.ShapeDtypeStruct((nn, D), jnp.float32),
            jax.ShapeDtypeStruct((1, D), jnp.float32),
        ],
        scratch_shapes=[pltpu.VMEM((1, D), jnp.float32)],
        interpret=_INTERPRET,
    )(n, sent, recv, vn, vs, vr, g, vg, b1.reshape(1, D), v2, b2.reshape(1, D))


def _glob_body(na_ref, ea_ref, g_ref, un_ref, ue_ref, ug_ref, b1_ref,
               u2_ref, b2_ref, o_ref):
    h = (
        jnp.dot(na_ref[...], un_ref[...], preferred_element_type=jnp.float32)
        + jnp.dot(ea_ref[...], ue_ref[...], preferred_element_type=jnp.float32)
        + jnp.dot(g_ref[...], ug_ref[...], preferred_element_type=jnp.float32)
        + b1_ref[...]
    )
    o_ref[...] = (
        jnp.dot(jnp.maximum(h, 0.0), u2_ref[...],
                preferred_element_type=jnp.float32)
        + b2_ref[...]
    )


def _glob_mlp(na, ea, g, un, ue, ug, b1, u2, b2):
    full = lambda: (0, 0)
    return pl.pallas_call(
        _glob_body,
        in_specs=[pl.BlockSpec(s, None) for s in
                  [(1, D), (1, D), (1, D), (D, D), (D, D), (D, D), (1, D),
                   (D, D), (1, D)]],
        out_specs=pl.BlockSpec((1, D), None),
        out_shape=jax.ShapeDtypeStruct((1, D), jnp.float32),
        interpret=_INTERPRET,
    )(na, ea, g, un, ue, ug, b1.reshape(1, D), u2, b2.reshape(1, D))


def _readout_body(g_ref, w1_ref, b1_ref, w2t_ref, b2_ref, o_ref):
    h = (
        jnp.dot(g_ref[...], w1_ref[...], preferred_element_type=jnp.float32)
        + b1_ref[...]
    )
    h = jnp.maximum(h, 0.0)
    o_ref[...] = (
        jnp.sum(h * w2t_ref[...], axis=1, keepdims=True) + b2_ref[...]
    )


def _readout(g, w1, b1, w2, b2):
    return pl.pallas_call(
        _readout_body,
        in_specs=[pl.BlockSpec(s, None) for s in
                  [(1, D), (D, D), (1, D), (1, D), (1, 1)]],
        out_specs=pl.BlockSpec((1, 1), None),
        out_shape=jax.ShapeDtypeStruct((1, 1), jnp.float32),
        interpret=_INTERPRET,
    )(g, w1, b1.reshape(1, D), w2.reshape(1, D), b2.reshape(1, 1))


_CH = 80
_CPW = N_EDGES // _CH // 16
N_PAD = 10240
_NSL = N_PAD // 16
_ZCH = 128
_SC_MESH = dict(core_axis_name="c", subcore_axis_name="s",
                num_cores=2, num_subcores=16)


def _gather_body(ps_hbm, pr_hbm, s3d, r3d, gs_hbm, gr_hbm, idx_v, rows_v, sem):
    c = lax.axis_index("c")
    ss = lax.axis_index("s")
    ebase = ss * _CPW * _CH

    @pl.when(c == 0)
    def _():
        pltpu.sync_copy(s3d.at[ss], idx_v)

        def body(k):
            pltpu.async_copy(ps_hbm.at[idx_v.at[k]], rows_v, sem).wait()
            pltpu.sync_copy(rows_v, gs_hbm.at[pl.ds(ebase + k * _CH, _CH)])

        pl.loop(0, _CPW)(body)

    @pl.when(c == 1)
    def _():
        pltpu.sync_copy(r3d.at[ss], idx_v)

        def body(k):
            pltpu.async_copy(pr_hbm.at[idx_v.at[k]], rows_v, sem).wait()
            pltpu.sync_copy(rows_v, gr_hbm.at[pl.ds(ebase + k * _CH, _CH)])

        pl.loop(0, _CPW)(body)


def _sc_gather(ps, pr, s3d, r3d):
    f = pl.kernel(
        _gather_body,
        out_type=[
            jax.ShapeDtypeStruct((N_EDGES, D), jnp.float32),
            jax.ShapeDtypeStruct((N_EDGES, D), jnp.float32),
        ],
        mesh=plsc.VectorSubcoreMesh(**_SC_MESH),
        scratch_types=[
            pltpu.VMEM((_CPW, _CH), jnp.int32),
            pltpu.VMEM((_CH, D), jnp.float32),
            pltpu.SemaphoreType.DMA,
        ],
    )
    return f(ps, pr, s3d, r3d)


def _scatter_body(e_hbm, s3d, r3d, sent_hbm, recv_hbm,
                  acc, idx_v, rows_v, zbuf):
    c = lax.axis_index("c")
    ss = lax.axis_index("s")
    slice_base = ss * _NSL

    def zrow(i):
        for j in range(8):
            zbuf[i, pl.ds(j * 16, 16)] = jnp.zeros((16,), jnp.float32)

    pl.loop(0, _ZCH)(zrow)

    def zcp(i):
        pltpu.sync_copy(zbuf, acc.at[pl.ds(slice_base + i * _ZCH, _ZCH)])

    pl.loop(0, _NSL // _ZCH)(zcp)
    plsc.subcore_barrier()

    @pl.when(c == 0)
    def _():
        pltpu.sync_copy(s3d.at[ss], idx_v)

    @pl.when(c == 1)
    def _():
        pltpu.sync_copy(r3d.at[ss], idx_v)

    ebase = ss * _CPW * _CH

    def body(k):
        pltpu.sync_copy(e_hbm.at[pl.ds(ebase + k * _CH, _CH)], rows_v)
        pltpu.sync_copy(rows_v, acc.at[idx_v.at[k]], add=True)

    pl.loop(0, _CPW)(body)
    plsc.subcore_barrier()

    def wcp(i):
        sl = pl.ds(slice_base + i * _ZCH, _ZCH)

        @pl.when(c == 0)
        def _():
            pltpu.sync_copy(acc.at[sl], sent_hbm.at[sl])

        @pl.when(c == 1)
        def _():
            pltpu.sync_copy(acc.at[sl], recv_hbm.at[sl])

    pl.loop(0, _NSL // _ZCH)(wcp)


def _sc_segment_sums(e, s3d, r3d):
    f = pl.kernel(
        _scatter_body,
        out_type=[
            jax.ShapeDtypeStruct((N_PAD, D), jnp.float32),
            jax.ShapeDtypeStruct((N_PAD, D), jnp.float32),
        ],
        mesh=plsc.VectorSubcoreMesh(**_SC_MESH),
        scratch_types=[
            pltpu.VMEM_SHARED((N_PAD, D), jnp.float32),
            pltpu.VMEM((_CPW, _CH), jnp.int32),
            pltpu.VMEM((_CH, D), jnp.float32),
            pltpu.VMEM((_ZCH, D), jnp.float32),
        ],
    )
    return f(e, s3d, r3d)


def _softplus(x):
    return jnp.log(1.0 + jnp.exp(x))


def _sample_mlp(layers, key):
    ks = jax.random.split(key, len(layers))
    out = []
    for p, k in zip(layers, ks):
        w = p['w_mu'] + jax.random.normal(k, p['w_mu'].shape,
                                          dtype=jnp.float32) * _softplus(p['w_rho'])
        b = p['b_mu'] + jax.random.normal(k, p['b_mu'].shape,
                                          dtype=jnp.float32) * _softplus(p['b_rho'])
        out.append((w, b))
    return out


def kernel(nodes, edges, senders, receivers, globals_, positions, box, params):
    keys = jax.random.split(jax.random.key(42), 4)
    emb = params['embed']

    n = _mm_bias(nodes, emb['node_w'], emb['node_b'], blk=2000)
    g = _mm_bias(globals_, emb['glob_w'], emb['glob_b'], blk=1)

    e = None
    zero_vec = jnp.zeros((1, D), jnp.float32)
    eye = jnp.eye(D, dtype=jnp.float32)
    s3d = senders.reshape(16, _CPW, _CH)
    r3d = receivers.reshape(16, _CPW, _CH)

    for s in range(2):
        sp = params['steps'][s]
        k_e, k_n, k_g = jax.random.split(keys[s], 3)
        (ew1, eb1), (ew2, eb2) = _sample_mlp(sp['edge'], k_e)
        (nw1, nb1), (nw2, nb2) = _sample_mlp(sp['node'], k_n)
        (gw1, gb1), (gw2, gb2) = _sample_mlp(sp['glob'], k_g)

        w1e = ew1[0:D]
        w1s = ew1[D:2 * D]
        w1r = ew1[2 * D:3 * D]
        w1g = ew1[3 * D:4 * D]

        if s == 0:
            ps, pr = _proj(n, w1s, w1r, g, w1g, eb1,
                           emb['edge_b'].reshape(1, D), w1e)
            x, wa, wb = edges, emb['edge_w'], w1e
        else:
            ps, pr = _proj(n, w1s, w1r, g, w1g, eb1, zero_vec, eye)
            x, wa, wb = e, w1e, eye

        gs, gr = _sc_gather(ps, pr, s3d, r3d)
        e, e_agg = _edge_mlp(x, gs, gr, wa, wb, ew2, eb2)
        sent, recv = _sc_segment_sums(e, s3d, r3d)

        n, n_agg = _node_mlp(n, sent, recv,
                             nw1[0:D], nw1[D:2 * D], nw1[2 * D:3 * D],
                             g, nw1[3 * D:4 * D], nb1, nw2, nb2)
        g = _glob_mlp(n_agg, e_agg, g,
                      gw1[0:D], gw1[D:2 * D], gw1[2 * D:3 * D], gb1,
                      gw2, gb2)

    (rw1, rb1), (rw2, rb2) = _sample_mlp(params['readout'], keys[-1])
    return _readout(g, rw1, rb1, rw2, rb2)


# speedup vs baseline: 3.6202x; 1.2623x over previous
import functools

import jax
import jax.numpy as jnp
from jax import lax
from jax.experimental import pallas as pl
from jax.experimental.pallas import tpu as pltpu
from jax.experimental.pallas import tpu_sc as plsc

_INTERPRET = False

N_NODES = 10000
N_EDGES = 160000
D = 128


def _mm_bias_body(x_ref, w_ref, b_ref, o_ref):
    o_ref[...] = (
        jnp.dot(x_ref[...], w_ref[...], preferred_element_type=jnp.float32)
        + b_ref[...]
    )


def _mm_bias(x, w, b, blk):
    n, k = x.shape
    m = w.shape[1]
    grid = n // blk
    return pl.pallas_call(
        _mm_bias_body,
        grid=(grid,),
        in_specs=[
            pl.BlockSpec((blk, k), lambda i: (i, 0)),
            pl.BlockSpec((k, m), lambda i: (0, 0)),
            pl.BlockSpec((1, m), lambda i: (0, 0)),
        ],
        out_specs=pl.BlockSpec((blk, m), lambda i: (i, 0)),
        out_shape=jax.ShapeDtypeStruct((n, m), jnp.float32),
        interpret=_INTERPRET,
    )(x, w, b.reshape(1, m))


def _proj_body(n_ref, ws_ref, wr_ref, g_ref, wg_ref, b1_ref, ev_ref, we_ref,
               ps_ref, pr_ref):
    c = (
        jnp.dot(g_ref[...], wg_ref[...], preferred_element_type=jnp.float32)
        + b1_ref[...]
        + jnp.dot(ev_ref[...], we_ref[...], preferred_element_type=jnp.float32)
    )
    nb = n_ref[...]
    ps_ref[...] = (
        jnp.dot(nb, ws_ref[...], preferred_element_type=jnp.float32) + c
    )
    pr_ref[...] = jnp.dot(nb, wr_ref[...], preferred_element_type=jnp.float32)


def _proj(n, w1s, w1r, g, w1g, b1, extra_vec, extra_mat, blk=2000):
    nn = n.shape[0]
    grid = nn // blk
    return pl.pallas_call(
        _proj_body,
        grid=(grid,),
        in_specs=[
            pl.BlockSpec((blk, D), lambda i: (i, 0)),
            pl.BlockSpec((D, D), lambda i: (0, 0)),
            pl.BlockSpec((D, D), lambda i: (0, 0)),
            pl.BlockSpec((1, D), lambda i: (0, 0)),
            pl.BlockSpec((D, D), lambda i: (0, 0)),
            pl.BlockSpec((1, D), lambda i: (0, 0)),
            pl.BlockSpec((1, extra_mat.shape[0]), lambda i: (0, 0)),
            pl.BlockSpec((extra_mat.shape[0], D), lambda i: (0, 0)),
        ],
        out_specs=[
            pl.BlockSpec((blk, D), lambda i: (i, 0)),
            pl.BlockSpec((blk, D), lambda i: (i, 0)),
        ],
        out_shape=[
            jax.ShapeDtypeStruct((nn, D), jnp.float32),
            jax.ShapeDtypeStruct((nn, D), jnp.float32),
        ],
        interpret=_INTERPRET,
    )(n, w1s, w1r, g, w1g, b1.reshape(1, D), extra_vec, extra_mat)


def _edge_body(x_ref, gs_ref, gr_ref, wa_ref, wb_ref, w2_ref, b2_ref,
               out_ref, agg_ref, acc_ref):
    a = jnp.dot(wa_ref[...], wb_ref[...], preferred_element_type=jnp.float32)
    h = (
        jnp.dot(x_ref[...], a, preferred_element_type=jnp.float32)
        + gs_ref[...]
        + gr_ref[...]
    )
    y = (
        jnp.dot(jnp.maximum(h, 0.0), w2_ref[...],
                preferred_element_type=jnp.float32)
        + b2_ref[...]
    )
    out_ref[...] = y
    i = pl.program_id(0)

    @pl.when(i == 0)
    def _():
        acc_ref[...] = jnp.zeros_like(acc_ref)

    acc_ref[...] += jnp.sum(y, axis=0, keepdims=True)

    @pl.when(i == pl.num_programs(0) - 1)
    def _():
        agg_ref[...] = acc_ref[...]


def _edge_mlp(x, gs_rows, gr_rows, wa, wb, w2, b2, blk=2000):
    ne, k = x.shape
    grid = ne // blk
    return pl.pallas_call(
        _edge_body,
        grid=(grid,),
        in_specs=[
            pl.BlockSpec((blk, k), lambda i: (i, 0)),
            pl.BlockSpec((blk, D), lambda i: (i, 0)),
            pl.BlockSpec((blk, D), lambda i: (i, 0)),
            pl.BlockSpec((k, D), lambda i: (0, 0)),
            pl.BlockSpec((D, D), lambda i: (0, 0)),
            pl.BlockSpec((D, D), lambda i: (0, 0)),
            pl.BlockSpec((1, D), lambda i: (0, 0)),
        ],
        out_specs=[
            pl.BlockSpec((blk, D), lambda i: (i, 0)),
            pl.BlockSpec((1, D), lambda i: (0, 0)),
        ],
        out_shape=[
            jax.ShapeDtypeStruct((ne, D), jnp.float32),
            jax.ShapeDtypeStruct((1, D), jnp.float32),
        ],
        scratch_shapes=[pltpu.VMEM((1, D), jnp.float32)],
        interpret=_INTERPRET,
    )(x, gs_rows, gr_rows, wa, wb, w2, b2.reshape(1, D))


def _node_body(n_ref, s_ref, r_ref, vn_ref, vs_ref, vr_ref, g_ref, vg_ref,
               b1_ref, v2_ref, b2_ref, out_ref, agg_ref, acc_ref):
    c = (
        jnp.dot(g_ref[...], vg_ref[...], preferred_element_type=jnp.float32)
        + b1_ref[...]
    )
    h = (
        jnp.dot(n_ref[...], vn_ref[...], preferred_element_type=jnp.float32)
        + jnp.dot(s_ref[...], vs_ref[...], preferred_element_type=jnp.float32)
        + jnp.dot(r_ref[...], vr_ref[...], preferred_element_type=jnp.float32)
        + c
    )
    y = (
        jnp.dot(jnp.maximum(h, 0.0), v2_ref[...],
                preferred_element_type=jnp.float32)
        + b2_ref[...]
    )
    out_ref[...] = y
    i = pl.program_id(0)

    @pl.when(i == 0)
    def _():
        acc_ref[...] = jnp.zeros_like(acc_ref)

    acc_ref[...] += jnp.sum(y, axis=0, keepdims=True)

    @pl.when(i == pl.num_programs(0) - 1)
    def _():
        agg_ref[...] = acc_ref[...]


def _node_mlp(n, sent, recv, vn, vs, vr, g, vg, b1, v2, b2, blk=2000):
    nn = n.shape[0]
    grid = nn // blk
    full = lambda i: (0, 0)
    rows = lambda i: (i, 0)
    return pl.pallas_call(
        _node_body,
        grid=(grid,),
        in_specs=[
            pl.BlockSpec((blk, D), rows),
            pl.BlockSpec((blk, D), rows),
            pl.BlockSpec((blk, D), rows),
            pl.BlockSpec((D, D), full),
            pl.BlockSpec((D, D), full),
            pl.BlockSpec((D, D), full),
            pl.BlockSpec((1, D), full),
            pl.BlockSpec((D, D), full),
            pl.BlockSpec((1, D), full),
            pl.BlockSpec((D, D), full),
            pl.BlockSpec((1, D), full),
        ],
        out_specs=[
            pl.BlockSpec((blk, D), rows),
            pl.BlockSpec((1, D), full),
        ],
        out_shape=[
            jax.ShapeDtypeStruct((nn, D), jnp.float32),
            jax.ShapeDtypeStruct((1, D), jnp.float32),
        ],
        scratch_shapes=[pltpu.VMEM((1, D), jnp.float32)],
        interpret=_INTERPRET,
    )(n, sent, recv, vn, vs, vr, g, vg, b1.reshape(1, D), v2, b2.reshape(1, D))


def _glob_body(na_ref, ea_ref, g_ref, un_ref, ue_ref, ug_ref, b1_ref,
               u2_ref, b2_ref, o_ref):
    h = (
        jnp.dot(na_ref[...], un_ref[...], preferred_element_type=jnp.float32)
        + jnp.dot(ea_ref[...], ue_ref[...], preferred_element_type=jnp.float32)
        + jnp.dot(g_ref[...], ug_ref[...], preferred_element_type=jnp.float32)
        + b1_ref[...]
    )
    o_ref[...] = (
        jnp.dot(jnp.maximum(h, 0.0), u2_ref[...],
                preferred_element_type=jnp.float32)
        + b2_ref[...]
    )


def _glob_mlp(na, ea, g, un, ue, ug, b1, u2, b2):
    full = lambda: (0, 0)
    return pl.pallas_call(
        _glob_body,
        in_specs=[pl.BlockSpec(s, None) for s in
                  [(1, D), (1, D), (1, D), (D, D), (D, D), (D, D), (1, D),
                   (D, D), (1, D)]],
        out_specs=pl.BlockSpec((1, D), None),
        out_shape=jax.ShapeDtypeStruct((1, D), jnp.float32),
        interpret=_INTERPRET,
    )(na, ea, g, un, ue, ug, b1.reshape(1, D), u2, b2.reshape(1, D))


def _readout_body(g_ref, w1_ref, b1_ref, w2t_ref, b2_ref, o_ref):
    h = (
        jnp.dot(g_ref[...], w1_ref[...], preferred_element_type=jnp.float32)
        + b1_ref[...]
    )
    h = jnp.maximum(h, 0.0)
    o_ref[...] = (
        jnp.sum(h * w2t_ref[...], axis=1, keepdims=True) + b2_ref[...]
    )


def _readout(g, w1, b1, w2, b2):
    return pl.pallas_call(
        _readout_body,
        in_specs=[pl.BlockSpec(s, None) for s in
                  [(1, D), (D, D), (1, D), (1, D), (1, 1)]],
        out_specs=pl.BlockSpec((1, 1), None),
        out_shape=jax.ShapeDtypeStruct((1, 1), jnp.float32),
        interpret=_INTERPRET,
    )(g, w1, b1.reshape(1, D), w2.reshape(1, D), b2.reshape(1, 1))


_CH = 80
_CPW = N_EDGES // _CH // 16
N_PAD = 10240
_NSL = N_PAD // 16
_ZCH = 32
_SC_MESH = dict(core_axis_name="c", subcore_axis_name="s",
                num_cores=2, num_subcores=16)


_NBUF = 5
_NRND = _CPW // _NBUF


def _gather_one(tab_hbm, idx3d, out_hbm, idx_v, rows, gsems, wsems, ss):
    ebase = ss * _CPW * _CH
    pltpu.sync_copy(idx3d.at[ss], idx_v)

    for b in range(_NBUF):
        pltpu.async_copy(tab_hbm.at[idx_v.at[b]], rows[b], gsems[b])

    def rnd(r):
        for b in range(_NBUF):
            k = r * _NBUF + b
            pltpu.make_async_copy(tab_hbm.at[idx_v.at[b]], rows[b],
                                  gsems[b]).wait()
            pltpu.async_copy(rows[b],
                             out_hbm.at[pl.ds(ebase + k * _CH, _CH)],
                             wsems[b])

        @pl.when(r < _NRND - 1)
        def _():
            for b in range(_NBUF):
                pltpu.make_async_copy(rows[b],
                                      out_hbm.at[pl.ds(ebase, _CH)],
                                      wsems[b]).wait()
                pltpu.async_copy(tab_hbm.at[idx_v.at[(r + 1) * _NBUF + b]],
                                 rows[b], gsems[b])

        @pl.when(r == _NRND - 1)
        def _():
            for b in range(_NBUF):
                pltpu.make_async_copy(rows[b],
                                      out_hbm.at[pl.ds(ebase, _CH)],
                                      wsems[b]).wait()

    pl.loop(0, _NRND)(rnd)


def _gather_body(ps_hbm, pr_hbm, s3d, r3d, gs_hbm, gr_hbm, idx_v,
                 r0, r1, r2, r3, r4, g0, g1, g2, g3, g4, w0, w1, w2, w3, w4):
    c = lax.axis_index("c")
    ss = lax.axis_index("s")
    rows = [r0, r1, r2, r3, r4]
    gsems = [g0, g1, g2, g3, g4]
    wsems = [w0, w1, w2, w3, w4]

    @pl.when(c == 0)
    def _():
        _gather_one(ps_hbm, s3d, gs_hbm, idx_v, rows, gsems, wsems, ss)

    @pl.when(c == 1)
    def _():
        _gather_one(pr_hbm, r3d, gr_hbm, idx_v, rows, gsems, wsems, ss)


def _sc_gather(ps, pr, s3d, r3d):
    f = pl.kernel(
        _gather_body,
        out_type=[
            jax.ShapeDtypeStruct((N_EDGES, D), jnp.float32),
            jax.ShapeDtypeStruct((N_EDGES, D), jnp.float32),
        ],
        mesh=plsc.VectorSubcoreMesh(**_SC_MESH),
        scratch_types=(
            [pltpu.VMEM((_CPW, _CH), jnp.int32)]
            + [pltpu.VMEM((_CH, D), jnp.float32) for _ in range(_NBUF)]
            + [pltpu.SemaphoreType.DMA for _ in range(2 * _NBUF)]
        ),
    )
    return f(ps, pr, s3d, r3d)


_SNB = 2
_SNR = (_CPW - 1) // _SNB


def _scatter_body(e_hbm, s3d, r3d, sent_hbm, recv_hbm,
                  acc, idx_v, zbuf, r0, r1, g0, g1, w0, w1):
    c = lax.axis_index("c")
    ss = lax.axis_index("s")
    slice_base = ss * _NSL
    rows = [r0, r1]
    rsems = [g0, g1]
    ssems = [w0, w1]

    def zrow(i):
        for j in range(8):
            zbuf[i, pl.ds(j * 16, 16)] = jnp.zeros((16,), jnp.float32)

    pl.loop(0, _ZCH)(zrow)

    def zcp(i):
        pltpu.sync_copy(zbuf, acc.at[pl.ds(slice_base + i * _ZCH, _ZCH)])

    pl.loop(0, _NSL // _ZCH)(zcp)

    @pl.when(c == 0)
    def _():
        pltpu.sync_copy(s3d.at[ss], idx_v)

    @pl.when(c == 1)
    def _():
        pltpu.sync_copy(r3d.at[ss], idx_v)

    plsc.subcore_barrier()

    ebase = ss * _CPW * _CH

    for b in range(_SNB):
        pltpu.async_copy(e_hbm.at[pl.ds(ebase + b * _CH, _CH)], rows[b],
                         rsems[b])

    def rnd(r):
        for b in range(_SNB):
            pltpu.make_async_copy(e_hbm.at[pl.ds(ebase, _CH)], rows[b],
                                  rsems[b]).wait()
            pltpu.async_copy(rows[b], acc.at[idx_v.at[r * _SNB + b]],
                             ssems[b], add=True)

        @pl.when(r < _SNR - 1)
        def _():
            for b in range(_SNB):
                pltpu.make_async_copy(rows[b], acc.at[idx_v.at[b]],
                                      ssems[b]).wait()
                k = (r + 1) * _SNB + b
                pltpu.async_copy(e_hbm.at[pl.ds(ebase + k * _CH, _CH)],
                                 rows[b], rsems[b])

        @pl.when(r == _SNR - 1)
        def _():
            for b in range(_SNB):
                pltpu.make_async_copy(rows[b], acc.at[idx_v.at[b]],
                                      ssems[b]).wait()

    pl.loop(0, _SNR)(rnd)

    pltpu.async_copy(e_hbm.at[pl.ds(ebase + (_CPW - 1) * _CH, _CH)],
                     rows[0], rsems[0])
    pltpu.make_async_copy(e_hbm.at[pl.ds(ebase, _CH)], rows[0],
                          rsems[0]).wait()
    pltpu.async_copy(rows[0], acc.at[idx_v.at[_CPW - 1]], ssems[0], add=True)
    pltpu.make_async_copy(rows[0], acc.at[idx_v.at[0]], ssems[0]).wait()

    plsc.subcore_barrier()

    def wcp(i):
        sl = pl.ds(slice_base + i * _ZCH, _ZCH)

        @pl.when(c == 0)
        def _():
            pltpu.sync_copy(acc.at[sl], sent_hbm.at[sl])

        @pl.when(c == 1)
        def _():
            pltpu.sync_copy(acc.at[sl], recv_hbm.at[sl])

    pl.loop(0, _NSL // _ZCH)(wcp)


def _sc_segment_sums(e, s3d, r3d):
    f = pl.kernel(
        _scatter_body,
        out_type=[
            jax.ShapeDtypeStruct((N_PAD, D), jnp.float32),
            jax.ShapeDtypeStruct((N_PAD, D), jnp.float32),
        ],
        mesh=plsc.VectorSubcoreMesh(**_SC_MESH),
        scratch_types=(
            [
                pltpu.VMEM_SHARED((N_PAD, D), jnp.float32),
                pltpu.VMEM((_CPW, _CH), jnp.int32),
                pltpu.VMEM((_ZCH, D), jnp.float32),
            ]
            + [pltpu.VMEM((_CH, D), jnp.float32) for _ in range(_SNB)]
            + [pltpu.SemaphoreType.DMA for _ in range(2 * _SNB)]
        ),
    )
    return f(e, s3d, r3d)


def _softplus(x):
    return jnp.log(1.0 + jnp.exp(x))


def _sample_mlp(layers, key):
    ks = jax.random.split(key, len(layers))
    out = []
    for p, k in zip(layers, ks):
        w = p['w_mu'] + jax.random.normal(k, p['w_mu'].shape,
                                          dtype=jnp.float32) * _softplus(p['w_rho'])
        b = p['b_mu'] + jax.random.normal(k, p['b_mu'].shape,
                                          dtype=jnp.float32) * _softplus(p['b_rho'])
        out.append((w, b))
    return out


def kernel(nodes, edges, senders, receivers, globals_, positions, box, params):
    keys = jax.random.split(jax.random.key(42), 4)
    emb = params['embed']

    n = _mm_bias(nodes, emb['node_w'], emb['node_b'], blk=2000)
    g = _mm_bias(globals_, emb['glob_w'], emb['glob_b'], blk=1)

    e = None
    zero_vec = jnp.zeros((1, D), jnp.float32)
    eye = jnp.eye(D, dtype=jnp.float32)
    s3d = senders.reshape(16, _CPW, _CH)
    r3d = receivers.reshape(16, _CPW, _CH)

    for s in range(2):
        sp = params['steps'][s]
        k_e, k_n, k_g = jax.random.split(keys[s], 3)
        (ew1, eb1), (ew2, eb2) = _sample_mlp(sp['edge'], k_e)
        (nw1, nb1), (nw2, nb2) = _sample_mlp(sp['node'], k_n)
        (gw1, gb1), (gw2, gb2) = _sample_mlp(sp['glob'], k_g)

        w1e = ew1[0:D]
        w1s = ew1[D:2 * D]
        w1r = ew1[2 * D:3 * D]
        w1g = ew1[3 * D:4 * D]

        if s == 0:
            ps, pr = _proj(n, w1s, w1r, g, w1g, eb1,
                           emb['edge_b'].reshape(1, D), w1e)
            x, wa, wb = edges, emb['edge_w'], w1e
        else:
            ps, pr = _proj(n, w1s, w1r, g, w1g, eb1, zero_vec, eye)
            x, wa, wb = e, w1e, eye

        gs, gr = _sc_gather(ps, pr, s3d, r3d)
        e, e_agg = _edge_mlp(x, gs, gr, wa, wb, ew2, eb2)
        sent, recv = _sc_segment_sums(e, s3d, r3d)

        n, n_agg = _node_mlp(n, sent, recv,
                             nw1[0:D], nw1[D:2 * D], nw1[2 * D:3 * D],
                             g, nw1[3 * D:4 * D], nb1, nw2, nb2)
        g = _glob_mlp(n_agg, e_agg, g,
                      gw1[0:D], gw1[D:2 * D], gw1[2 * D:3 * D], gb1,
                      gw2, gb2)

    (rw1, rb1), (rw2, rb2) = _sample_mlp(params['readout'], keys[-1])
    return _readout(g, rw1, rb1, rw2, rb2)


# speedup vs baseline: 4.0428x; 1.1167x over previous
import functools

import jax
import jax.numpy as jnp
from jax import lax
from jax.experimental import pallas as pl
from jax.experimental.pallas import tpu as pltpu
from jax.experimental.pallas import tpu_sc as plsc

_INTERPRET = False

N_NODES = 10000
N_EDGES = 160000
D = 128


def _mm_bias_body(x_ref, w_ref, b_ref, o_ref):
    o_ref[...] = (
        jnp.dot(x_ref[...], w_ref[...], preferred_element_type=jnp.float32)
        + b_ref[...]
    )


def _mm_bias(x, w, b, blk):
    n, k = x.shape
    m = w.shape[1]
    grid = n // blk
    return pl.pallas_call(
        _mm_bias_body,
        grid=(grid,),
        in_specs=[
            pl.BlockSpec((blk, k), lambda i: (i, 0)),
            pl.BlockSpec((k, m), lambda i: (0, 0)),
            pl.BlockSpec((1, m), lambda i: (0, 0)),
        ],
        out_specs=pl.BlockSpec((blk, m), lambda i: (i, 0)),
        out_shape=jax.ShapeDtypeStruct((n, m), jnp.float32),
        interpret=_INTERPRET,
    )(x, w, b.reshape(1, m))


def _embed_nodes_body(x_ref, w_ref, b_ref, ws_ref, wr_ref,
                      n_ref, ps_ref, pr_ref):
    n = (
        jnp.dot(x_ref[...], w_ref[...], preferred_element_type=jnp.float32)
        + b_ref[...]
    )
    n_ref[...] = n
    ps_ref[...] = jnp.dot(n, ws_ref[...], preferred_element_type=jnp.float32)
    pr_ref[...] = jnp.dot(n, wr_ref[...], preferred_element_type=jnp.float32)


def _embed_nodes(x, w, b, w1s, w1r, blk=2000):
    nn = x.shape[0]
    grid = nn // blk
    full = lambda i: (0, 0)
    rows = lambda i: (i, 0)
    return pl.pallas_call(
        _embed_nodes_body,
        grid=(grid,),
        in_specs=[
            pl.BlockSpec((blk, D), rows),
            pl.BlockSpec((D, D), full),
            pl.BlockSpec((1, D), full),
            pl.BlockSpec((D, D), full),
            pl.BlockSpec((D, D), full),
        ],
        out_specs=[pl.BlockSpec((blk, D), rows)] * 3,
        out_shape=[jax.ShapeDtypeStruct((nn, D), jnp.float32)] * 3,
        interpret=_INTERPRET,
    )(x, w, b.reshape(1, D), w1s, w1r)


def _edge_body(x_ref, gs_ref, gr_ref, wa_ref, wb_ref, w2_ref, b2_ref,
               g_ref, wg_ref, b1_ref, ev_ref, em_ref,
               out_ref, agg_ref, acc_ref):
    a = jnp.dot(wa_ref[...], wb_ref[...], preferred_element_type=jnp.float32)
    c = (
        jnp.dot(g_ref[...], wg_ref[...], preferred_element_type=jnp.float32)
        + b1_ref[...]
        + jnp.dot(ev_ref[...], em_ref[...], preferred_element_type=jnp.float32)
    )
    h = (
        jnp.dot(x_ref[...], a, preferred_element_type=jnp.float32)
        + gs_ref[...]
        + gr_ref[...]
        + c
    )
    y = (
        jnp.dot(jnp.maximum(h, 0.0), w2_ref[...],
                preferred_element_type=jnp.float32)
        + b2_ref[...]
    )
    out_ref[...] = y
    i = pl.program_id(0)

    @pl.when(i == 0)
    def _():
        acc_ref[...] = jnp.zeros_like(acc_ref)

    acc_ref[...] += jnp.sum(y, axis=0, keepdims=True)

    @pl.when(i == pl.num_programs(0) - 1)
    def _():
        agg_ref[...] = acc_ref[...]


def _edge_mlp(x, gs_rows, gr_rows, wa, wb, w2, b2, g, wg, b1, ev, em,
              blk=2000):
    ne, k = x.shape
    grid = ne // blk
    full = lambda i: (0, 0)
    rows = lambda i: (i, 0)
    return pl.pallas_call(
        _edge_body,
        grid=(grid,),
        in_specs=[
            pl.BlockSpec((blk, k), rows),
            pl.BlockSpec((blk, D), rows),
            pl.BlockSpec((blk, D), rows),
            pl.BlockSpec((k, D), full),
            pl.BlockSpec((D, D), full),
            pl.BlockSpec((D, D), full),
            pl.BlockSpec((1, D), full),
            pl.BlockSpec((1, D), full),
            pl.BlockSpec((D, D), full),
            pl.BlockSpec((1, D), full),
            pl.BlockSpec((1, D), full),
            pl.BlockSpec((D, D), full),
        ],
        out_specs=[
            pl.BlockSpec((blk, D), rows),
            pl.BlockSpec((1, D), full),
        ],
        out_shape=[
            jax.ShapeDtypeStruct((ne, D), jnp.float32),
            jax.ShapeDtypeStruct((1, D), jnp.float32),
        ],
        scratch_shapes=[pltpu.VMEM((1, D), jnp.float32)],
        interpret=_INTERPRET,
    )(x, gs_rows, gr_rows, wa, wb, w2, b2.reshape(1, D),
      g, wg, b1.reshape(1, D), ev, em)


def _node_body_proj(n_ref, s_ref, r_ref, vn_ref, vs_ref, vr_ref, g_ref,
                    vg_ref, b1_ref, v2_ref, b2_ref, ws_ref, wr_ref,
                    out_ref, agg_ref, ps_ref, pr_ref, acc_ref):
    c = (
        jnp.dot(g_ref[...], vg_ref[...], preferred_element_type=jnp.float32)
        + b1_ref[...]
    )
    h = (
        jnp.dot(n_ref[...], vn_ref[...], preferred_element_type=jnp.float32)
        + jnp.dot(s_ref[...], vs_ref[...], preferred_element_type=jnp.float32)
        + jnp.dot(r_ref[...], vr_ref[...], preferred_element_type=jnp.float32)
        + c
    )
    y = (
        jnp.dot(jnp.maximum(h, 0.0), v2_ref[...],
                preferred_element_type=jnp.float32)
        + b2_ref[...]
    )
    out_ref[...] = y
    if ps_ref is not None:
        ps_ref[...] = jnp.dot(y, ws_ref[...],
                              preferred_element_type=jnp.float32)
        pr_ref[...] = jnp.dot(y, wr_ref[...],
                              preferred_element_type=jnp.float32)
    i = pl.program_id(0)

    @pl.when(i == 0)
    def _():
        acc_ref[...] = jnp.zeros_like(acc_ref)

    acc_ref[...] += jnp.sum(y, axis=0, keepdims=True)

    @pl.when(i == pl.num_programs(0) - 1)
    def _():
        agg_ref[...] = acc_ref[...]


def _node_body_noproj(n_ref, s_ref, r_ref, vn_ref, vs_ref, vr_ref, g_ref,
                      vg_ref, b1_ref, v2_ref, b2_ref,
                      out_ref, agg_ref, acc_ref):
    _node_body_proj(n_ref, s_ref, r_ref, vn_ref, vs_ref, vr_ref, g_ref,
                    vg_ref, b1_ref, v2_ref, b2_ref, None, None,
                    out_ref, agg_ref, None, None, acc_ref)


def _node_mlp(n, sent, recv, vn, vs, vr, g, vg, b1, v2, b2,
              ws_next=None, wr_next=None, blk=2000):
    nn = n.shape[0]
    grid = nn // blk
    full = lambda i: (0, 0)
    rows = lambda i: (i, 0)
    with_proj = ws_next is not None
    in_specs = [
        pl.BlockSpec((blk, D), rows),
        pl.BlockSpec((blk, D), rows),
        pl.BlockSpec((blk, D), rows),
        pl.BlockSpec((D, D), full),
        pl.BlockSpec((D, D), full),
        pl.BlockSpec((D, D), full),
        pl.BlockSpec((1, D), full),
        pl.BlockSpec((D, D), full),
        pl.BlockSpec((1, D), full),
        pl.BlockSpec((D, D), full),
        pl.BlockSpec((1, D), full),
    ]
    out_specs = [pl.BlockSpec((blk, D), rows), pl.BlockSpec((1, D), full)]
    out_shape = [
        jax.ShapeDtypeStruct((nn, D), jnp.float32),
        jax.ShapeDtypeStruct((1, D), jnp.float32),
    ]
    args = [n, sent, recv, vn, vs, vr, g, vg, b1.reshape(1, D), v2,
            b2.reshape(1, D)]
    if with_proj:
        in_specs += [pl.BlockSpec((D, D), full)] * 2
        out_specs += [pl.BlockSpec((blk, D), rows)] * 2
        out_shape += [jax.ShapeDtypeStruct((nn, D), jnp.float32)] * 2
        args += [ws_next, wr_next]
    return pl.pallas_call(
        _node_body_proj if with_proj else _node_body_noproj,
        grid=(grid,),
        in_specs=in_specs,
        out_specs=out_specs,
        out_shape=out_shape,
        scratch_shapes=[pltpu.VMEM((1, D), jnp.float32)],
        interpret=_INTERPRET,
    )(*args)


def _glob_body(na_ref, ea_ref, g_ref, un_ref, ue_ref, ug_ref, b1_ref,
               u2_ref, b2_ref, o_ref):
    h = (
        jnp.dot(na_ref[...], un_ref[...], preferred_element_type=jnp.float32)
        + jnp.dot(ea_ref[...], ue_ref[...], preferred_element_type=jnp.float32)
        + jnp.dot(g_ref[...], ug_ref[...], preferred_element_type=jnp.float32)
        + b1_ref[...]
    )
    o_ref[...] = (
        jnp.dot(jnp.maximum(h, 0.0), u2_ref[...],
                preferred_element_type=jnp.float32)
        + b2_ref[...]
    )


def _glob_mlp(na, ea, g, un, ue, ug, b1, u2, b2):
    full = lambda: (0, 0)
    return pl.pallas_call(
        _glob_body,
        in_specs=[pl.BlockSpec(s, None) for s in
                  [(1, D), (1, D), (1, D), (D, D), (D, D), (D, D), (1, D),
                   (D, D), (1, D)]],
        out_specs=pl.BlockSpec((1, D), None),
        out_shape=jax.ShapeDtypeStruct((1, D), jnp.float32),
        interpret=_INTERPRET,
    )(na, ea, g, un, ue, ug, b1.reshape(1, D), u2, b2.reshape(1, D))


def _readout_body(g_ref, w1_ref, b1_ref, w2t_ref, b2_ref, o_ref):
    h = (
        jnp.dot(g_ref[...], w1_ref[...], preferred_element_type=jnp.float32)
        + b1_ref[...]
    )
    h = jnp.maximum(h, 0.0)
    o_ref[...] = (
        jnp.sum(h * w2t_ref[...], axis=1, keepdims=True) + b2_ref[...]
    )


def _readout(g, w1, b1, w2, b2):
    return pl.pallas_call(
        _readout_body,
        in_specs=[pl.BlockSpec(s, None) for s in
                  [(1, D), (D, D), (1, D), (1, D), (1, 1)]],
        out_specs=pl.BlockSpec((1, 1), None),
        out_shape=jax.ShapeDtypeStruct((1, 1), jnp.float32),
        interpret=_INTERPRET,
    )(g, w1, b1.reshape(1, D), w2.reshape(1, D), b2.reshape(1, 1))


_CH = 80
_CPW = N_EDGES // _CH // 16
N_PAD = 10240
_NSL = N_PAD // 16
_ZCH = 32
_SC_MESH = dict(core_axis_name="c", subcore_axis_name="s",
                num_cores=2, num_subcores=16)


_NBUF = 5
_NRND = _CPW // _NBUF


def _gather_one(tab_hbm, idx3d, out_hbm, idx_v, rows, gsems, wsems, ss):
    ebase = ss * _CPW * _CH
    pltpu.sync_copy(idx3d.at[ss], idx_v)

    for b in range(_NBUF):
        pltpu.async_copy(tab_hbm.at[idx_v.at[b]], rows[b], gsems[b])

    def rnd(r):
        for b in range(_NBUF):
            k = r * _NBUF + b
            pltpu.make_async_copy(tab_hbm.at[idx_v.at[b]], rows[b],
                                  gsems[b]).wait()
            pltpu.async_copy(rows[b],
                             out_hbm.at[pl.ds(ebase + k * _CH, _CH)],
                             wsems[b])

        @pl.when(r < _NRND - 1)
        def _():
            for b in range(_NBUF):
                pltpu.make_async_copy(rows[b],
                                      out_hbm.at[pl.ds(ebase, _CH)],
                                      wsems[b]).wait()
                pltpu.async_copy(tab_hbm.at[idx_v.at[(r + 1) * _NBUF + b]],
                                 rows[b], gsems[b])

        @pl.when(r == _NRND - 1)
        def _():
            for b in range(_NBUF):
                pltpu.make_async_copy(rows[b],
                                      out_hbm.at[pl.ds(ebase, _CH)],
                                      wsems[b]).wait()

    pl.loop(0, _NRND)(rnd)


def _gather_body(ps_hbm, pr_hbm, s3d, r3d, gs_hbm, gr_hbm, idx_v,
                 r0, r1, r2, r3, r4, g0, g1, g2, g3, g4, w0, w1, w2, w3, w4):
    c = lax.axis_index("c")
    ss = lax.axis_index("s")
    rows = [r0, r1, r2, r3, r4]
    gsems = [g0, g1, g2, g3, g4]
    wsems = [w0, w1, w2, w3, w4]

    @pl.when(c == 0)
    def _():
        _gather_one(ps_hbm, s3d, gs_hbm, idx_v, rows, gsems, wsems, ss)

    @pl.when(c == 1)
    def _():
        _gather_one(pr_hbm, r3d, gr_hbm, idx_v, rows, gsems, wsems, ss)


def _sc_gather(ps, pr, s3d, r3d):
    f = pl.kernel(
        _gather_body,
        out_type=[
            jax.ShapeDtypeStruct((N_EDGES, D), jnp.float32),
            jax.ShapeDtypeStruct((N_EDGES, D), jnp.float32),
        ],
        mesh=plsc.VectorSubcoreMesh(**_SC_MESH),
        scratch_types=(
            [pltpu.VMEM((_CPW, _CH), jnp.int32)]
            + [pltpu.VMEM((_CH, D), jnp.float32) for _ in range(_NBUF)]
            + [pltpu.SemaphoreType.DMA for _ in range(2 * _NBUF)]
        ),
    )
    return f(ps, pr, s3d, r3d)


_SNB = 3
_SNR = (_CPW - 2) // _SNB


def _scatter_body(e_hbm, s3d, r3d, sent_hbm, recv_hbm,
                  acc, idx_v, r0, r1, r2, g0, g1, g2, w0, w1, w2):
    c = lax.axis_index("c")
    ss = lax.axis_index("s")
    slice_base = ss * _NSL
    rows = [r0, r1, r2]
    rsems = [g0, g1, g2]
    ssems = [w0, w1, w2]

    def zrow(i):
        for j in range(8):
            r0[i, pl.ds(j * 16, 16)] = jnp.zeros((16,), jnp.float32)

    pl.loop(0, _CH)(zrow)

    def zcp(i):
        pltpu.sync_copy(r0, acc.at[pl.ds(slice_base + i * _CH, _CH)])

    pl.loop(0, _NSL // _CH)(zcp)

    @pl.when(c == 0)
    def _():
        pltpu.sync_copy(s3d.at[ss], idx_v)

    @pl.when(c == 1)
    def _():
        pltpu.sync_copy(r3d.at[ss], idx_v)

    plsc.subcore_barrier()

    ebase = ss * _CPW * _CH

    for b in range(_SNB):
        pltpu.async_copy(e_hbm.at[pl.ds(ebase + b * _CH, _CH)], rows[b],
                         rsems[b])

    def rnd(r):
        for b in range(_SNB):
            pltpu.make_async_copy(e_hbm.at[pl.ds(ebase, _CH)], rows[b],
                                  rsems[b]).wait()
            pltpu.async_copy(rows[b], acc.at[idx_v.at[r * _SNB + b]],
                             ssems[b], add=True)

        @pl.when(r < _SNR - 1)
        def _():
            for b in range(_SNB):
                pltpu.make_async_copy(rows[b], acc.at[idx_v.at[b]],
                                      ssems[b]).wait()
                k = (r + 1) * _SNB + b
                pltpu.async_copy(e_hbm.at[pl.ds(ebase + k * _CH, _CH)],
                                 rows[b], rsems[b])

        @pl.when(r == _SNR - 1)
        def _():
            for b in range(_SNB):
                pltpu.make_async_copy(rows[b], acc.at[idx_v.at[b]],
                                      ssems[b]).wait()

    pl.loop(0, _SNR)(rnd)

    for t in range(2):
        k = _CPW - 2 + t
        pltpu.async_copy(e_hbm.at[pl.ds(ebase + k * _CH, _CH)],
                         rows[t], rsems[t])
    for t in range(2):
        k = _CPW - 2 + t
        pltpu.make_async_copy(e_hbm.at[pl.ds(ebase, _CH)], rows[t],
                              rsems[t]).wait()
        pltpu.async_copy(rows[t], acc.at[idx_v.at[k]], ssems[t], add=True)
    for t in range(2):
        pltpu.make_async_copy(rows[t], acc.at[idx_v.at[0]], ssems[t]).wait()

    plsc.subcore_barrier()

    def wcp(i):
        sl = pl.ds(slice_base + i * _CH, _CH)

        @pl.when(c == 0)
        def _():
            pltpu.sync_copy(acc.at[sl], sent_hbm.at[sl])

        @pl.when(c == 1)
        def _():
            pltpu.sync_copy(acc.at[sl], recv_hbm.at[sl])

    pl.loop(0, _NSL // _CH)(wcp)


def _sc_segment_sums(e, s3d, r3d):
    f = pl.kernel(
        _scatter_body,
        out_type=[
            jax.ShapeDtypeStruct((N_PAD, D), jnp.float32),
            jax.ShapeDtypeStruct((N_PAD, D), jnp.float32),
        ],
        mesh=plsc.VectorSubcoreMesh(**_SC_MESH),
        scratch_types=(
            [
                pltpu.VMEM_SHARED((N_PAD, D), jnp.float32),
                pltpu.VMEM((_CPW, _CH), jnp.int32),
            ]
            + [pltpu.VMEM((_CH, D), jnp.float32) for _ in range(_SNB)]
            + [pltpu.SemaphoreType.DMA for _ in range(2 * _SNB)]
        ),
    )
    return f(e, s3d, r3d)


def _softplus(x):
    return jnp.log(1.0 + jnp.exp(x))


def _sample_mlp(layers, key):
    ks = jax.random.split(key, len(layers))
    out = []
    for p, k in zip(layers, ks):
        w = p['w_mu'] + jax.random.normal(k, p['w_mu'].shape,
                                          dtype=jnp.float32) * _softplus(p['w_rho'])
        b = p['b_mu'] + jax.random.normal(k, p['b_mu'].shape,
                                          dtype=jnp.float32) * _softplus(p['b_rho'])
        out.append((w, b))
    return out


def kernel(nodes, edges, senders, receivers, globals_, positions, box, params):
    keys = jax.random.split(jax.random.key(42), 4)
    emb = params['embed']

    zero_vec = jnp.zeros((1, D), jnp.float32)
    eye = jnp.eye(D, dtype=jnp.float32)
    s3d = senders.reshape(16, _CPW, _CH)
    r3d = receivers.reshape(16, _CPW, _CH)

    sw = []
    for s in range(2):
        sp = params['steps'][s]
        k_e, k_n, k_g = jax.random.split(keys[s], 3)
        sw.append((_sample_mlp(sp['edge'], k_e),
                   _sample_mlp(sp['node'], k_n),
                   _sample_mlp(sp['glob'], k_g)))
    (rw1, rb1), (rw2, rb2) = _sample_mlp(params['readout'], keys[-1])

    ew1_0 = sw[0][0][0][0]
    n, ps, pr = _embed_nodes(nodes, emb['node_w'], emb['node_b'],
                             ew1_0[D:2 * D], ew1_0[2 * D:3 * D])
    g = _mm_bias(globals_, emb['glob_w'], emb['glob_b'], blk=1)

    e = None
    for s in range(2):
        (ew1, eb1), (ew2, eb2) = sw[s][0]
        (nw1, nb1), (nw2, nb2) = sw[s][1]
        (gw1, gb1), (gw2, gb2) = sw[s][2]
        w1e = ew1[0:D]
        w1g = ew1[3 * D:4 * D]

        if s == 0:
            x, wa, wb = edges, emb['edge_w'], w1e
            ev, em = emb['edge_b'].reshape(1, D), w1e
        else:
            x, wa, wb = e, w1e, eye
            ev, em = zero_vec, eye

        gs, gr = _sc_gather(ps, pr, s3d, r3d)
        e, e_agg = _edge_mlp(x, gs, gr, wa, wb, ew2, eb2,
                             g, w1g, eb1, ev, em)
        sent, recv = _sc_segment_sums(e, s3d, r3d)

        if s == 0:
            ew1_n = sw[1][0][0][0]
            n, n_agg, ps, pr = _node_mlp(
                n, sent, recv,
                nw1[0:D], nw1[D:2 * D], nw1[2 * D:3 * D],
                g, nw1[3 * D:4 * D], nb1, nw2, nb2,
                ws_next=ew1_n[D:2 * D], wr_next=ew1_n[2 * D:3 * D])
        else:
            n, n_agg = _node_mlp(
                n, sent, recv,
                nw1[0:D], nw1[D:2 * D], nw1[2 * D:3 * D],
                g, nw1[3 * D:4 * D], nb1, nw2, nb2)
        g = _glob_mlp(n_agg, e_agg, g,
                      gw1[0:D], gw1[D:2 * D], gw1[2 * D:3 * D], gb1,
                      gw2, gb2)

    return _readout(g, rw1, rb1, rw2, rb2)
